# baseline (speedup 1.0000x reference)
import functools

import jax
import jax.numpy as jnp
from jax import lax
from jax.experimental import pallas as pl
from jax.experimental.pallas import tpu as pltpu
from jax.experimental.pallas import tpu_sc as plsc

F32 = jnp.float32
LN_EPS = 1e-5

TN = 2048
TE = 2048
CHUNK = 128
NC = 2
NS = 16
NW = NC * NS


def _rup(x, m):
    return (x + m - 1) // m * m


def _ln(h, s, t):
    mu = jnp.mean(h, axis=-1, keepdims=True)
    var = jnp.mean((h - mu) * (h - mu), axis=-1, keepdims=True)
    return (h - mu) * lax.rsqrt(var + LN_EPS) * s + t


BF16 = jnp.bfloat16


def _pack_bf16(x):
    lo = lax.bitcast_convert_type(
        x[:, :64].astype(BF16), jnp.uint16).astype(jnp.uint32)
    hi = lax.bitcast_convert_type(
        x[:, 64:].astype(BF16), jnp.uint16).astype(jnp.uint32)
    return lax.bitcast_convert_type(lo | (hi << 16), F32)


def _unpack_bf16(w):
    wu = lax.bitcast_convert_type(w, jnp.uint32)
    lo = lax.bitcast_convert_type(
        (wu & 0xFFFF).astype(jnp.uint16), BF16).astype(F32)
    hi = lax.bitcast_convert_type(
        (wu >> 16).astype(jnp.uint16), BF16).astype(F32)
    return jnp.concatenate([lo, hi], axis=-1)


def _enc_node_body(x, w0, b0, w1, b1, w2, b2, s, t, ws, wd, o_hn, o_t):
    h = jnp.maximum(x[...] @ w0[...] + b0[...], 0.0)
    h = jnp.maximum(h @ w1[...] + b1[...], 0.0)
    h = h @ w2[...] + b2[...]
    hn = _ln(h, s[...], t[...])
    o_hn[...] = hn
    o_t[...] = jnp.concatenate(
        [_pack_bf16(hn @ ws[...]), _pack_bf16(hn @ wd[...])], axis=-1)


def _enc_edge_body(x, w0, b0, w1, b1, w2, b2, s, t, o):
    h = jnp.maximum(x[...] @ w0[...] + b0[...], 0.0)
    h = jnp.maximum(h @ w1[...] + b1[...], 0.0)
    h = h @ w2[...] + b2[...]
    o[...] = _ln(h, s[...], t[...])


def _edge_body(he, g_pk, w0e, b0, w1, b1, w2, b2, s, t, o):
    x = he[...]
    gv = g_pk[...]
    g = _unpack_bf16(gv[:, :64]) + _unpack_bf16(gv[:, 64:])
    h = jnp.maximum(x @ w0e[...] + g + b0[...], 0.0)
    h = jnp.maximum(h @ w1[...] + b1[...], 0.0)
    h = h @ w2[...] + b2[...]
    o[...] = _ln(h, s[...], t[...]) + x


def _node_body(p0, p1, hn, w0a, w0h, b0, w1, b1, w2, b2, s, t, ws, wd,
               o_hn, o_t):
    x = hn[...]
    agg = p0[...] + p1[...]
    h = jnp.maximum(agg @ w0a[...] + x @ w0h[...] + b0[...], 0.0)
    h = jnp.maximum(h @ w1[...] + b1[...], 0.0)
    h = h @ w2[...] + b2[...]
    hn2 = _ln(h, s[...], t[...]) + x
    o_hn[...] = hn2
    o_t[...] = jnp.concatenate(
        [_pack_bf16(hn2 @ ws[...]), _pack_bf16(hn2 @ wd[...])], axis=-1)


def _dec_body(x, w0, b0, w1, b1, w2, b2, o):
    h = jnp.maximum(x[...] @ w0[...] + b0[...], 0.0)
    h = jnp.maximum(h @ w1[...] + b1[...], 0.0)
    o[...] = h @ w2[...] + b2[...]


def _row_spec(tile, width):
    return pl.BlockSpec((tile, width), lambda i: (i, 0))


def _full_spec(shape):
    return pl.BlockSpec(shape, lambda i: (0,) * len(shape))


def _mat(shape=(128, 128)):
    return _full_spec(shape)


def _vec():
    return _full_spec((1, 128))


def _tc_call(body, grid, in_specs, out_specs, out_shape):
    return pl.pallas_call(
        body,
        grid=(grid,),
        in_specs=in_specs,
        out_specs=out_specs,
        out_shape=out_shape,
    )


def _sc_gather(tab, src3, dst3, e_pad):
    per_w = e_pad // NW
    nchunk = per_w // CHUNK
    mesh = plsc.VectorSubcoreMesh(core_axis_name="c", subcore_axis_name="s", num_cores=NC, num_subcores=NS)

    @functools.partial(
        pl.kernel,
        mesh=mesh,
        out_type=jax.ShapeDtypeStruct((e_pad, 128), F32),
        scratch_types=[
            pltpu.VMEM((nchunk, CHUNK), jnp.int32),
            pltpu.VMEM((nchunk, CHUNK), jnp.int32),
            pltpu.VMEM((2, CHUNK, 128), F32),
            pltpu.VMEM((2, CHUNK, 128), F32),
        ] + [pltpu.SemaphoreType.DMA] * 6,
    )
    def k(t_hbm, src_hbm, dst_hbm, g_hbm,
          isv, idv, bd, bc, g0, g1, g2, g3, w0, w1):
        wid = lax.axis_index("s") * NC + lax.axis_index("c")
        base = wid * per_w
        pltpu.sync_copy(src_hbm.at[wid], isv)
        pltpu.sync_copy(dst_hbm.at[wid], idv)

        def body(j, carry):
            c0 = 2 * j
            c1 = c0 + 1
            ga0 = pltpu.async_copy(t_hbm.at[isv.at[c0]], bc.at[0], g0)
            gb0 = pltpu.async_copy(t_hbm.at[idv.at[c0]], bd.at[0], g1)
            ga1 = pltpu.async_copy(t_hbm.at[isv.at[c1]], bc.at[1], g2)
            gb1 = pltpu.async_copy(t_hbm.at[idv.at[c1]], bd.at[1], g3)
            o0 = base + c0 * CHUNK
            o1 = base + c1 * CHUNK

            def merge(r, buf):
                for q in range(4):
                    col = pl.ds(64 + 16 * q, 16)
                    bc[buf, r, col] = bd[buf, r, col]
                return buf

            ga0.wait()
            gb0.wait()
            lax.fori_loop(0, CHUNK, merge, 0)
            wa0 = pltpu.async_copy(bc.at[0], g_hbm.at[pl.ds(o0, CHUNK)], w0)
            ga1.wait()
            gb1.wait()
            lax.fori_loop(0, CHUNK, merge, 1)
            wa1 = pltpu.async_copy(bc.at[1], g_hbm.at[pl.ds(o1, CHUNK)], w1)
            wa0.wait()
            wa1.wait()
            return carry

        lax.fori_loop(0, nchunk // 2, body, 0)

    return k(tab, src3, dst3)


def _sc_scatter(he, dst3, zeros, e_pad, n_pad):
    per_w = e_pad // NW
    nchunk = per_w // CHUNK
    rows_per_sub = n_pad // NS
    mesh = plsc.VectorSubcoreMesh(core_axis_name="c", subcore_axis_name="s", num_cores=NC, num_subcores=NS)

    @functools.partial(
        pl.kernel,
        mesh=mesh,
        out_type=(jax.ShapeDtypeStruct((n_pad, 128), F32),
                  jax.ShapeDtypeStruct((n_pad, 128), F32)),
        scratch_types=[
            pltpu.VMEM((nchunk, CHUNK), jnp.int32),
            pltpu.VMEM((2, CHUNK, 128), F32),
            pltpu.VMEM_SHARED((n_pad, 128), F32),
            pltpu.SemaphoreType.DMA,
            pltpu.SemaphoreType.DMA,
        ],
    )
    def k(he_hbm, dst_hbm, z_hbm, p0_hbm, p1_hbm, idx_v, rows_v, acc_sh,
          sem0, sem1):
        cid = lax.axis_index("c")
        sid = lax.axis_index("s")
        sl = pl.ds(sid * rows_per_sub, rows_per_sub)
        pltpu.sync_copy(z_hbm.at[sl], acc_sh.at[sl])
        plsc.subcore_barrier()

        wid = sid * NC + cid
        base = wid * per_w
        pltpu.sync_copy(dst_hbm.at[wid], idx_v)

        def body(j, carry):
            c0 = 2 * j
            c1 = c0 + 1
            l0 = pltpu.async_copy(
                he_hbm.at[pl.ds(base + c0 * CHUNK, CHUNK)], rows_v.at[0], sem0)
            l1 = pltpu.async_copy(
                he_hbm.at[pl.ds(base + c1 * CHUNK, CHUNK)], rows_v.at[1], sem1)
            l0.wait()
            pltpu.sync_copy(rows_v.at[0], acc_sh.at[idx_v.at[c0]], add=True)
            l1.wait()
            pltpu.sync_copy(rows_v.at[1], acc_sh.at[idx_v.at[c1]], add=True)
            return carry

        lax.fori_loop(0, nchunk // 2, body, 0)
        plsc.subcore_barrier()

        @pl.when(cid == 0)
        def _():
            pltpu.sync_copy(acc_sh.at[sl], p0_hbm.at[sl])

        @pl.when(cid == 1)
        def _():
            pltpu.sync_copy(acc_sh.at[sl], p1_hbm.at[sl])

    return k(he, dst3, zeros)


def kernel(node_features, edge_features, edge_index, params):
    n, d_node = node_features.shape
    e, d_edge = edge_features.shape
    p_blocks = len(params["edge_blocks"])
    n_pad = _rup(n, max(TN, NS * 8))
    e_pad = _rup(e, max(TE, NW * CHUNK))
    gn = n_pad // TN
    ge = e_pad // TE

    x_n = jnp.pad(node_features, ((0, n_pad - n), (0, 0)))
    perm = jnp.argsort(edge_index[1])
    x_e = jnp.pad(edge_features[perm], ((0, e_pad - e), (0, 0)))
    nchunk = e_pad // NW // CHUNK
    src3 = jnp.pad(edge_index[0][perm],
                   (0, e_pad - e)).reshape(NW, nchunk, CHUNK)
    dst3 = jnp.pad(edge_index[1][perm], (0, e_pad - e),
                   constant_values=n).reshape(NW, nchunk, CHUNK)
    zeros = jnp.zeros((n_pad, 128), F32)

    def vec(v):
        return v.reshape(1, -1)

    def mlp_args(p):
        return (p["w0"], vec(p["b0"]), p["w1"], vec(p["b1"]),
                p["w2"], vec(p["b2"]))

    def ln_args(p):
        return (vec(p["ln_s"]), vec(p["ln_b"]))

    def esplit(p):
        w0 = p["w0"]
        return w0[:128], w0[128:256], w0[256:384]

    def nsplit(p):
        w0 = p["w0"]
        return w0[:128], w0[128:256]

    hw = jax.ShapeDtypeStruct((n_pad, 128), F32)
    ht = jax.ShapeDtypeStruct((n_pad, 128), F32)
    ew = jax.ShapeDtypeStruct((e_pad, 128), F32)

    pe = params["enc_e"]
    he = _tc_call(
        _enc_edge_body, ge,
        [_row_spec(TE, d_edge), _mat((d_edge, 128)), _vec(), _mat(), _vec(),
         _mat(), _vec(), _vec(), _vec()],
        _row_spec(TE, 128), ew,
    )(x_e, *mlp_args(pe), *ln_args(pe))

    pn = params["enc_n"]
    ws0, wd0 = esplit(params["edge_blocks"][0])[1:]
    hn, tab = _tc_call(
        _enc_node_body, gn,
        [_row_spec(TN, d_node), _mat((d_node, 128)), _vec(), _mat(), _vec(),
         _mat(), _vec(), _vec(), _vec(), _mat(), _mat()],
        (_row_spec(TN, 128),) * 2, (hw, ht),
    )(x_n, *mlp_args(pn), *ln_args(pn), ws0, wd0)

    for i in range(p_blocks):
        pe_i = params["edge_blocks"][i]
        w0e = esplit(pe_i)[0]
        g_pk = _sc_gather(tab, src3, dst3, e_pad)
        he = _tc_call(
            _edge_body, ge,
            [_row_spec(TE, 128), _row_spec(TE, 128)] +
            [_mat(), _vec(), _mat(), _vec(), _mat(), _vec(), _vec(), _vec()],
            _row_spec(TE, 128), ew,
        )(he, g_pk, w0e, *mlp_args(pe_i)[1:], *ln_args(pe_i))

        p0, p1 = _sc_scatter(he, dst3, zeros, e_pad, n_pad)

        pn_i = params["node_blocks"][i]
        w0a, w0h = nsplit(pn_i)
        if i + 1 < p_blocks:
            ws_n, wd_n = esplit(params["edge_blocks"][i + 1])[1:]
        else:
            ws_n = wd_n = jnp.zeros((128, 128), F32)
        hn, tab = _tc_call(
            _node_body, gn,
            [_row_spec(TN, 128)] * 3 +
            [_mat(), _mat(), _vec(), _mat(), _vec(), _mat(), _vec(),
             _vec(), _vec(), _mat(), _mat()],
            (_row_spec(TN, 128),) * 2, (hw, ht),
        )(p0, p1, hn, w0a, w0h, *mlp_args(pn_i)[1:], *ln_args(pn_i),
          ws_n, wd_n)

    pd = params["dec"]
    dout = pd["w2"].shape[1]
    w2p = jnp.zeros((128, 128), F32).at[:, :dout].set(pd["w2"])
    b2p = jnp.zeros((1, 128), F32).at[0, :dout].set(pd["b2"])
    out = _tc_call(
        _dec_body, gn,
        [_row_spec(TN, 128), _mat(), _vec(), _mat(), _vec(), _mat(), _vec()],
        _row_spec(TN, 128), hw,
    )(hn, pd["w0"], vec(pd["b0"]), pd["w1"], vec(pd["b1"]), w2p, b2p)

    return out[:n, :dout]


# speedup vs baseline: 1.0320x; 1.0320x over previous
import functools

import jax
import jax.numpy as jnp
from jax import lax
from jax.experimental import pallas as pl
from jax.experimental.pallas import tpu as pltpu
from jax.experimental.pallas import tpu_sc as plsc

F32 = jnp.float32
LN_EPS = 1e-5

TN = 2048
TE = 2048
CHUNK = 128
NC = 2
NS = 16
NW = NC * NS


def _rup(x, m):
    return (x + m - 1) // m * m


def _ln(h, s, t):
    mu = jnp.mean(h, axis=-1, keepdims=True)
    var = jnp.mean((h - mu) * (h - mu), axis=-1, keepdims=True)
    return (h - mu) * lax.rsqrt(var + LN_EPS) * s + t


BF16 = jnp.bfloat16


def _pack_bf16(x):
    lo = lax.bitcast_convert_type(
        x[:, :64].astype(BF16), jnp.uint16).astype(jnp.uint32)
    hi = lax.bitcast_convert_type(
        x[:, 64:].astype(BF16), jnp.uint16).astype(jnp.uint32)
    return lax.bitcast_convert_type(lo | (hi << 16), F32)


def _unpack_bf16(w):
    wu = lax.bitcast_convert_type(w, jnp.uint32)
    lo = lax.bitcast_convert_type(
        (wu & 0xFFFF).astype(jnp.uint16), BF16).astype(F32)
    hi = lax.bitcast_convert_type(
        (wu >> 16).astype(jnp.uint16), BF16).astype(F32)
    return jnp.concatenate([lo, hi], axis=-1)


def _enc_node_body(x, w0, b0, w1, b1, w2, b2, s, t, ws, wd, o_hn, o_t):
    h = jnp.maximum(x[...] @ w0[...] + b0[...], 0.0)
    h = jnp.maximum(h @ w1[...] + b1[...], 0.0)
    h = h @ w2[...] + b2[...]
    hn = _ln(h, s[...], t[...])
    o_hn[...] = hn
    o_t[...] = jnp.concatenate(
        [_pack_bf16(hn @ ws[...]), _pack_bf16(hn @ wd[...])], axis=-1)


def _enc_edge_body(x, w0, b0, w1, b1, w2, b2, s, t, o):
    h = jnp.maximum(x[...] @ w0[...] + b0[...], 0.0)
    h = jnp.maximum(h @ w1[...] + b1[...], 0.0)
    h = h @ w2[...] + b2[...]
    o[...] = _ln(h, s[...], t[...])


def _edge_body(he, g_pk, w0e, b0, w1, b1, w2, b2, s, t, o):
    x = he[...]
    gv = g_pk[...]
    g = _unpack_bf16(gv[:, :64]) + _unpack_bf16(gv[:, 64:])
    h = jnp.maximum(x @ w0e[...] + g + b0[...], 0.0)
    h = jnp.maximum(h @ w1[...] + b1[...], 0.0)
    h = h @ w2[...] + b2[...]
    o[...] = _ln(h, s[...], t[...]) + x


def _node_body(p0, p1, hn, w0a, w0h, b0, w1, b1, w2, b2, s, t, ws, wd,
               o_hn, o_t):
    x = hn[...]
    agg = p0[...] + p1[...]
    h = jnp.maximum(agg @ w0a[...] + x @ w0h[...] + b0[...], 0.0)
    h = jnp.maximum(h @ w1[...] + b1[...], 0.0)
    h = h @ w2[...] + b2[...]
    hn2 = _ln(h, s[...], t[...]) + x
    o_hn[...] = hn2
    o_t[...] = jnp.concatenate(
        [_pack_bf16(hn2 @ ws[...]), _pack_bf16(hn2 @ wd[...])], axis=-1)


def _dec_body(x, w0, b0, w1, b1, w2, b2, o):
    h = jnp.maximum(x[...] @ w0[...] + b0[...], 0.0)
    h = jnp.maximum(h @ w1[...] + b1[...], 0.0)
    o[...] = h @ w2[...] + b2[...]


def _row_spec(tile, width):
    return pl.BlockSpec((tile, width), lambda i: (i, 0))


def _full_spec(shape):
    return pl.BlockSpec(shape, lambda i: (0,) * len(shape))


def _mat(shape=(128, 128)):
    return _full_spec(shape)


def _vec():
    return _full_spec((1, 128))


def _tc_call(body, grid, in_specs, out_specs, out_shape):
    return pl.pallas_call(
        body,
        grid=(grid,),
        in_specs=in_specs,
        out_specs=out_specs,
        out_shape=out_shape,
    )


def _sc_gather(tab, src3, dst3, e_pad):
    per_w = e_pad // NW
    nchunk = per_w // CHUNK
    mesh = plsc.VectorSubcoreMesh(core_axis_name="c", subcore_axis_name="s", num_cores=NC, num_subcores=NS)

    @functools.partial(
        pl.kernel,
        mesh=mesh,
        out_type=jax.ShapeDtypeStruct((e_pad, 128), F32),
        scratch_types=[
            pltpu.VMEM((nchunk, CHUNK), jnp.int32),
            pltpu.VMEM((nchunk, CHUNK), jnp.int32),
            pltpu.VMEM((2, CHUNK, 128), F32),
            pltpu.VMEM((2, CHUNK, 128), F32),
        ] + [pltpu.SemaphoreType.DMA] * 6,
    )
    def k(t_hbm, src_hbm, dst_hbm, g_hbm,
          isv, idv, bd, bc, g0, g1, g2, g3, w0, w1):
        wid = lax.axis_index("s") * NC + lax.axis_index("c")
        base = wid * per_w
        pltpu.sync_copy(src_hbm.at[wid], isv)
        pltpu.sync_copy(dst_hbm.at[wid], idv)

        def body(j, carry):
            c0 = 2 * j
            c1 = c0 + 1
            ga0 = pltpu.async_copy(t_hbm.at[isv.at[c0]], bc.at[0], g0)
            gb0 = pltpu.async_copy(t_hbm.at[idv.at[c0]], bd.at[0], g1)
            ga1 = pltpu.async_copy(t_hbm.at[isv.at[c1]], bc.at[1], g2)
            gb1 = pltpu.async_copy(t_hbm.at[idv.at[c1]], bd.at[1], g3)
            o0 = base + c0 * CHUNK
            o1 = base + c1 * CHUNK

            def merge(i, buf):
                for p in range(4):
                    r = 4 * i + p
                    for q in range(4):
                        col = pl.ds(64 + 16 * q, 16)
                        bc[buf, r, col] = bd[buf, r, col]
                return buf

            ga0.wait()
            gb0.wait()
            lax.fori_loop(0, CHUNK // 4, merge, 0)
            wa0 = pltpu.async_copy(bc.at[0], g_hbm.at[pl.ds(o0, CHUNK)], w0)
            ga1.wait()
            gb1.wait()
            lax.fori_loop(0, CHUNK // 4, merge, 1)
            wa1 = pltpu.async_copy(bc.at[1], g_hbm.at[pl.ds(o1, CHUNK)], w1)
            wa0.wait()
            wa1.wait()
            return carry

        lax.fori_loop(0, nchunk // 2, body, 0)

    return k(tab, src3, dst3)


def _sc_scatter(he, dst3, zeros, e_pad, n_pad):
    per_w = e_pad // NW
    nchunk = per_w // CHUNK
    rows_per_sub = n_pad // NS
    mesh = plsc.VectorSubcoreMesh(core_axis_name="c", subcore_axis_name="s", num_cores=NC, num_subcores=NS)

    @functools.partial(
        pl.kernel,
        mesh=mesh,
        out_type=(jax.ShapeDtypeStruct((n_pad, 128), F32),
                  jax.ShapeDtypeStruct((n_pad, 128), F32)),
        scratch_types=[
            pltpu.VMEM((nchunk, CHUNK), jnp.int32),
            pltpu.VMEM((2, CHUNK, 128), F32),
            pltpu.VMEM_SHARED((n_pad, 128), F32),
            pltpu.SemaphoreType.DMA,
            pltpu.SemaphoreType.DMA,
        ],
    )
    def k(he_hbm, dst_hbm, z_hbm, p0_hbm, p1_hbm, idx_v, rows_v, acc_sh,
          sem0, sem1):
        cid = lax.axis_index("c")
        sid = lax.axis_index("s")
        sl = pl.ds(sid * rows_per_sub, rows_per_sub)
        pltpu.sync_copy(z_hbm.at[sl], acc_sh.at[sl])
        plsc.subcore_barrier()

        wid = sid * NC + cid
        base = wid * per_w
        pltpu.sync_copy(dst_hbm.at[wid], idx_v)

        def body(j, carry):
            c0 = 2 * j
            c1 = c0 + 1
            l0 = pltpu.async_copy(
                he_hbm.at[pl.ds(base + c0 * CHUNK, CHUNK)], rows_v.at[0], sem0)
            l1 = pltpu.async_copy(
                he_hbm.at[pl.ds(base + c1 * CHUNK, CHUNK)], rows_v.at[1], sem1)
            l0.wait()
            pltpu.sync_copy(rows_v.at[0], acc_sh.at[idx_v.at[c0]], add=True)
            l1.wait()
            pltpu.sync_copy(rows_v.at[1], acc_sh.at[idx_v.at[c1]], add=True)
            return carry

        lax.fori_loop(0, nchunk // 2, body, 0)
        plsc.subcore_barrier()

        @pl.when(cid == 0)
        def _():
            pltpu.sync_copy(acc_sh.at[sl], p0_hbm.at[sl])

        @pl.when(cid == 1)
        def _():
            pltpu.sync_copy(acc_sh.at[sl], p1_hbm.at[sl])

    return k(he, dst3, zeros)


def kernel(node_features, edge_features, edge_index, params):
    n, d_node = node_features.shape
    e, d_edge = edge_features.shape
    p_blocks = len(params["edge_blocks"])
    n_pad = _rup(n, max(TN, NS * 8))
    e_pad = _rup(e, max(TE, NW * CHUNK))
    gn = n_pad // TN
    ge = e_pad // TE

    x_n = jnp.pad(node_features, ((0, n_pad - n), (0, 0)))
    x_e = jnp.pad(edge_features, ((0, e_pad - e), (0, 0)))
    nchunk = e_pad // NW // CHUNK
    src3 = jnp.pad(edge_index[0], (0, e_pad - e)).reshape(NW, nchunk, CHUNK)
    dst3 = jnp.pad(edge_index[1], (0, e_pad - e),
                   constant_values=n).reshape(NW, nchunk, CHUNK)
    zeros = jnp.zeros((n_pad, 128), F32)

    def vec(v):
        return v.reshape(1, -1)

    def mlp_args(p):
        return (p["w0"], vec(p["b0"]), p["w1"], vec(p["b1"]),
                p["w2"], vec(p["b2"]))

    def ln_args(p):
        return (vec(p["ln_s"]), vec(p["ln_b"]))

    def esplit(p):
        w0 = p["w0"]
        return w0[:128], w0[128:256], w0[256:384]

    def nsplit(p):
        w0 = p["w0"]
        return w0[:128], w0[128:256]

    hw = jax.ShapeDtypeStruct((n_pad, 128), F32)
    ht = jax.ShapeDtypeStruct((n_pad, 128), F32)
    ew = jax.ShapeDtypeStruct((e_pad, 128), F32)

    pe = params["enc_e"]
    he = _tc_call(
        _enc_edge_body, ge,
        [_row_spec(TE, d_edge), _mat((d_edge, 128)), _vec(), _mat(), _vec(),
         _mat(), _vec(), _vec(), _vec()],
        _row_spec(TE, 128), ew,
    )(x_e, *mlp_args(pe), *ln_args(pe))

    pn = params["enc_n"]
    ws0, wd0 = esplit(params["edge_blocks"][0])[1:]
    hn, tab = _tc_call(
        _enc_node_body, gn,
        [_row_spec(TN, d_node), _mat((d_node, 128)), _vec(), _mat(), _vec(),
         _mat(), _vec(), _vec(), _vec(), _mat(), _mat()],
        (_row_spec(TN, 128),) * 2, (hw, ht),
    )(x_n, *mlp_args(pn), *ln_args(pn), ws0, wd0)

    for i in range(p_blocks):
        pe_i = params["edge_blocks"][i]
        w0e = esplit(pe_i)[0]
        g_pk = _sc_gather(tab, src3, dst3, e_pad)
        he = _tc_call(
            _edge_body, ge,
            [_row_spec(TE, 128), _row_spec(TE, 128)] +
            [_mat(), _vec(), _mat(), _vec(), _mat(), _vec(), _vec(), _vec()],
            _row_spec(TE, 128), ew,
        )(he, g_pk, w0e, *mlp_args(pe_i)[1:], *ln_args(pe_i))

        p0, p1 = _sc_scatter(he, dst3, zeros, e_pad, n_pad)

        pn_i = params["node_blocks"][i]
        w0a, w0h = nsplit(pn_i)
        if i + 1 < p_blocks:
            ws_n, wd_n = esplit(params["edge_blocks"][i + 1])[1:]
        else:
            ws_n = wd_n = jnp.zeros((128, 128), F32)
        hn, tab = _tc_call(
            _node_body, gn,
            [_row_spec(TN, 128)] * 3 +
            [_mat(), _mat(), _vec(), _mat(), _vec(), _mat(), _vec(),
             _vec(), _vec(), _mat(), _mat()],
            (_row_spec(TN, 128),) * 2, (hw, ht),
        )(p0, p1, hn, w0a, w0h, *mlp_args(pn_i)[1:], *ln_args(pn_i),
          ws_n, wd_n)

    pd = params["dec"]
    dout = pd["w2"].shape[1]
    w2p = jnp.zeros((128, 128), F32).at[:, :dout].set(pd["w2"])
    b2p = jnp.zeros((1, 128), F32).at[0, :dout].set(pd["b2"])
    out = _tc_call(
        _dec_body, gn,
        [_row_spec(TN, 128), _mat(), _vec(), _mat(), _vec(), _mat(), _vec()],
        _row_spec(TN, 128), hw,
    )(hn, pd["w0"], vec(pd["b0"]), pd["w1"], vec(pd["b1"]), w2p, b2p)

    return out[:n, :dout]
